# Initial kernel scaffold; baseline (speedup 1.0000x reference)
import jax
import jax.numpy as jnp
from jax.experimental import pallas as pl


def kernel(inputs, targets, alpha):
    raise NotImplementedError("write your pallas kernel here")


# speedup vs baseline: 2.8802x; 2.8802x over previous
import functools

import jax
import jax.numpy as jnp
from jax import lax
from jax.experimental import pallas as pl
from jax.experimental.pallas import tpu as pltpu
from jax.experimental.pallas import tpu_sc as plsc

_NC = 2
_NS = 16
_NW = _NC * _NS
_L = 16


def _focal_partials(n_rows, x_flat_hbm, tgt_hbm, alpha_hbm, out_hbm,
                    xbuf, tbuf, avec):
    rpw = n_rows // _NW
    wid = lax.axis_index("s") * _NC + lax.axis_index("c")
    row0 = wid * rpw

    pltpu.sync_copy(x_flat_hbm.at[pl.ds(row0 * 2, rpw * 2)], xbuf)
    pltpu.sync_copy(tgt_hbm.at[pl.ds(row0, rpw)], tbuf)
    pltpu.sync_copy(alpha_hbm, avec)

    lanes2 = 2 * lax.iota(jnp.int32, _L)
    zero = jnp.zeros((_L,), jnp.float32)

    def group(i, acc):
        base = i * _L
        t = tbuf[pl.ds(base, _L)]
        idx_t = lanes2 + (2 * base + t)
        idx_o = jnp.bitwise_xor(idx_t, 1)
        x_t = plsc.load_gather(xbuf, [idx_t])
        x_o = plsc.load_gather(xbuf, [idx_o])
        d = x_t - x_o
        e = jnp.exp(-jnp.abs(d))
        r = 1.0 / (1.0 + e)
        omp = jnp.where(d >= 0.0, e, 1.0) * r
        z = e / (2.0 + e)
        z2 = z * z
        l1p = z * (2.0 + z2 * (2.0 / 3.0 + z2 * (2.0 / 5.0 + z2 *
                   (2.0 / 7.0 + z2 * (2.0 / 9.0)))))
        nlp = jnp.maximum(-d, 0.0) + l1p
        return acc + omp * omp * nlp

    acc = lax.fori_loop(0, rpw // _L, group, zero)

    scale = jnp.sum(avec[...]) * (0.5 / n_rows)
    avec[...] = acc * scale
    pltpu.sync_copy(avec, out_hbm.at[pl.ds(wid * _L, _L)])


def kernel(inputs, targets, alpha):
    n = inputs.shape[0]
    alpha16 = jnp.pad(alpha.astype(jnp.float32),
                      (0, _L - alpha.shape[0]))
    partials = pl.kernel(
        functools.partial(_focal_partials, n),
        out_type=jax.ShapeDtypeStruct((_NW * _L,), jnp.float32),
        mesh=plsc.VectorSubcoreMesh(core_axis_name="c", subcore_axis_name="s"),
        compiler_params=pltpu.CompilerParams(needs_layout_passes=False),
        scratch_types=[
            pltpu.VMEM((n // _NW * 2,), jnp.float32),
            pltpu.VMEM((n // _NW,), jnp.int32),
            pltpu.VMEM((_L,), jnp.float32),
        ],
    )(inputs.reshape(-1), targets.astype(jnp.int32), alpha16)
    return jnp.sum(partials)


# speedup vs baseline: 80.3316x; 27.8908x over previous
import functools

import jax
import jax.numpy as jnp
from jax import lax
from jax.experimental import pallas as pl
from jax.experimental.pallas import tpu as pltpu
from jax.experimental.pallas import tpu_sc as plsc

_NC = 2
_NS = 16
_NW = _NC * _NS
_L = 16


def _focal_partials(n_rows, x0_hbm, x1_hbm, tgt_hbm, alpha_hbm, out_hbm,
                    x0buf, x1buf, tbuf, avec):
    rpw = n_rows // _NW
    wid = lax.axis_index("s") * _NC + lax.axis_index("c")
    row0 = wid * rpw

    pltpu.sync_copy(x0_hbm.at[pl.ds(row0, rpw)], x0buf)
    pltpu.sync_copy(x1_hbm.at[pl.ds(row0, rpw)], x1buf)
    pltpu.sync_copy(tgt_hbm.at[pl.ds(row0, rpw)], tbuf)
    pltpu.sync_copy(alpha_hbm, avec)

    zero = jnp.zeros((_L,), jnp.float32)

    def group(i, acc):
        base = i * _L
        t = tbuf[pl.ds(base, _L)]
        x0 = x0buf[pl.ds(base, _L)]
        x1 = x1buf[pl.ds(base, _L)]
        dx = x1 - x0
        d = jnp.where(t == 1, dx, -dx)
        e = jnp.exp(-jnp.abs(d))
        r = 1.0 / (1.0 + e)
        omp = jnp.where(d >= 0.0, e, 1.0) * r
        z = e / (2.0 + e)
        z2 = z * z
        l1p = z * (2.0 + z2 * (2.0 / 3.0 + z2 * (2.0 / 5.0 + z2 *
                   (2.0 / 7.0 + z2 * (2.0 / 9.0)))))
        nlp = jnp.maximum(-d, 0.0) + l1p
        return acc + omp * omp * nlp

    acc = lax.fori_loop(0, rpw // _L, group, zero)

    scale = jnp.sum(avec[...]) * (0.5 / n_rows)
    avec[...] = acc * scale
    pltpu.sync_copy(avec, out_hbm.at[pl.ds(wid * _L, _L)])


def kernel(inputs, targets, alpha):
    n = inputs.shape[0]
    alpha16 = jnp.pad(alpha.astype(jnp.float32),
                      (0, _L - alpha.shape[0]))
    partials = pl.kernel(
        functools.partial(_focal_partials, n),
        out_type=jax.ShapeDtypeStruct((_NW * _L,), jnp.float32),
        mesh=plsc.VectorSubcoreMesh(core_axis_name="c", subcore_axis_name="s"),
        compiler_params=pltpu.CompilerParams(needs_layout_passes=False),
        scratch_types=[
            pltpu.VMEM((n // _NW,), jnp.float32),
            pltpu.VMEM((n // _NW,), jnp.float32),
            pltpu.VMEM((n // _NW,), jnp.int32),
            pltpu.VMEM((_L,), jnp.float32),
        ],
    )(inputs[:, 0], inputs[:, 1], targets.astype(jnp.int32), alpha16)
    return jnp.sum(partials)
